# baseline (speedup 1.0000x reference)
import functools

import jax
import jax.numpy as jnp
from jax import lax
from jax.experimental import pallas as pl
from jax.experimental.pallas import tpu as pltpu
from jax.experimental.pallas import tpu_sc as plsc

_N = 10000
_E = 320000
_H = 128
_G = 64
_T = 32
_L = 4

_NPAD = 10240
_NW = 32
_WIN = 128
_NWIN = 80
_PERW = _NWIN * _WIN
_EPAD = _NW * _PERW
_ACC = 10368
_ZR = _ACC // 16


def _silu(v):
    return v * jax.nn.sigmoid(v)


def _in_body(x_ref, w_ref, b_ref, o_ref):
    o_ref[...] = _silu(
        jnp.dot(x_ref[...], w_ref[...], preferred_element_type=jnp.float32)
        + b_ref[...])


def _node_in(x, W, b):
    blk = 512
    n = x.shape[0]
    return pl.pallas_call(
        _in_body,
        grid=(n // blk,),
        in_specs=[
            pl.BlockSpec((blk, _H), lambda i: (i, 0)),
            pl.BlockSpec((_H, _H), lambda i: (0, 0)),
            pl.BlockSpec((1, _H), lambda i: (0, 0)),
        ],
        out_specs=pl.BlockSpec((blk, _H), lambda i: (i, 0)),
        out_shape=jax.ShapeDtypeStruct((n, _H), jnp.float32),
    )(x, W, b)


def _pq_body(f_ref, wa_ref, wb_ref, b_ref, p_ref, q_ref):
    fb = f_ref[...]
    p_ref[...] = jnp.dot(fb, wa_ref[...], preferred_element_type=jnp.float32)
    q_ref[...] = (jnp.dot(fb, wb_ref[...], preferred_element_type=jnp.float32)
                  + b_ref[...])


def _pq(f, Wa, Wb, b1):
    blk = 512
    n = f.shape[0]
    return pl.pallas_call(
        _pq_body,
        grid=(n // blk,),
        in_specs=[
            pl.BlockSpec((blk, _H), lambda i: (i, 0)),
            pl.BlockSpec((_H, _H), lambda i: (0, 0)),
            pl.BlockSpec((_H, _H), lambda i: (0, 0)),
            pl.BlockSpec((1, _H), lambda i: (0, 0)),
        ],
        out_specs=[
            pl.BlockSpec((blk, _H), lambda i: (i, 0)),
            pl.BlockSpec((blk, _H), lambda i: (i, 0)),
        ],
        out_shape=[
            jax.ShapeDtypeStruct((n, _H), jnp.float32),
            jax.ShapeDtypeStruct((n, _H), jnp.float32),
        ],
    )(f, Wa, Wb, b1)


def _edge_body(pg_ref, qg_ref, ew_ref, wc_ref, w2_ref, b2_ref, o_ref):
    sq = ew_ref[...] ** 2
    m1 = _silu(pg_ref[...] + qg_ref[...] + sq * wc_ref[...])
    m2 = _silu(
        jnp.dot(m1, w2_ref[...], preferred_element_type=jnp.float32)
        + b2_ref[...])
    o_ref[...] = _silu(m2)


def _edge(Pg, Qg, ew, wc, W2, b2):
    blk = 512
    ne = Pg.shape[0]
    return pl.pallas_call(
        _edge_body,
        grid=(ne // blk,),
        in_specs=[
            pl.BlockSpec((blk, _H), lambda i: (i, 0)),
            pl.BlockSpec((blk, _H), lambda i: (i, 0)),
            pl.BlockSpec((blk, 1), lambda i: (i, 0)),
            pl.BlockSpec((1, _H), lambda i: (0, 0)),
            pl.BlockSpec((_H, _H), lambda i: (0, 0)),
            pl.BlockSpec((1, _H), lambda i: (0, 0)),
        ],
        out_specs=pl.BlockSpec((blk, _H), lambda i: (i, 0)),
        out_shape=jax.ShapeDtypeStruct((ne, _H), jnp.float32),
    )(Pg, Qg, ew, wc, W2, b2)


def _upd_body(f_ref, a0_ref, a1_ref, b0_ref, b1p_ref, w1_ref, b1_ref,
              w2_ref, b2_ref, wa_ref, wb_ref, bm_ref,
              o_ref, p_ref, q_ref):
    fb = f_ref[...]
    u = a0_ref[0] + a1_ref[0] + b0_ref[0] + b1p_ref[0] + fb
    h = _silu(
        jnp.dot(u, w1_ref[...], preferred_element_type=jnp.float32)
        + b1_ref[...])
    fn = fb + (
        jnp.dot(h, w2_ref[...], preferred_element_type=jnp.float32)
        + b2_ref[...])
    o_ref[...] = fn
    p_ref[...] = jnp.dot(fn, wa_ref[...], preferred_element_type=jnp.float32)
    q_ref[...] = (jnp.dot(fn, wb_ref[...], preferred_element_type=jnp.float32)
                  + bm_ref[...])


def _upd(f, partsA, partsB, W1, b1, W2, b2, Wa, Wb, bm):
    blk = 512
    n = f.shape[0]
    return pl.pallas_call(
        _upd_body,
        grid=(n // blk,),
        in_specs=[
            pl.BlockSpec((blk, _H), lambda i: (i, 0)),
            pl.BlockSpec((1, blk, _H), lambda i: (0, i, 0)),
            pl.BlockSpec((1, blk, _H), lambda i: (1, i, 0)),
            pl.BlockSpec((1, blk, _H), lambda i: (0, i, 0)),
            pl.BlockSpec((1, blk, _H), lambda i: (1, i, 0)),
            pl.BlockSpec((_H, _H), lambda i: (0, 0)),
            pl.BlockSpec((1, _H), lambda i: (0, 0)),
            pl.BlockSpec((_H, _H), lambda i: (0, 0)),
            pl.BlockSpec((1, _H), lambda i: (0, 0)),
            pl.BlockSpec((_H, _H), lambda i: (0, 0)),
            pl.BlockSpec((_H, _H), lambda i: (0, 0)),
            pl.BlockSpec((1, _H), lambda i: (0, 0)),
        ],
        out_specs=[
            pl.BlockSpec((blk, _H), lambda i: (i, 0)),
            pl.BlockSpec((blk, _H), lambda i: (i, 0)),
            pl.BlockSpec((blk, _H), lambda i: (i, 0)),
        ],
        out_shape=[
            jax.ShapeDtypeStruct((n, _H), jnp.float32),
            jax.ShapeDtypeStruct((n, _H), jnp.float32),
            jax.ShapeDtypeStruct((n, _H), jnp.float32),
        ],
    )(f, partsA, partsA, partsB, partsB, W1, b1, W2, b2, Wa, Wb, bm)


def _ro_body(f_ref, gid_ref, w1_ref, b1_ref, w2_ref, b2_ref, gs_ref, cnt_ref):
    i = pl.program_id(0)
    f2 = _silu(
        jnp.dot(f_ref[...], w1_ref[...], preferred_element_type=jnp.float32)
        + b1_ref[...])
    f2 = (jnp.dot(f2, w2_ref[...], preferred_element_type=jnp.float32)
          + b2_ref[...])
    ids = gid_ref[0, 0, :]
    blk = ids.shape[0]
    onehot = (ids[:, None]
              == lax.broadcasted_iota(jnp.int32, (blk, _G), 1)
              ).astype(jnp.float32)
    partial = lax.dot_general(onehot, f2, (((0,), (0,)), ((), ())),
                              preferred_element_type=jnp.float32)
    pcnt = lax.dot_general(onehot, jnp.ones((blk, _H), jnp.float32),
                           (((0,), (0,)), ((), ())),
                           preferred_element_type=jnp.float32)

    @pl.when(i == 0)
    def _():
        gs_ref[...] = jnp.zeros_like(gs_ref)
        cnt_ref[...] = jnp.zeros_like(cnt_ref)

    gs_ref[...] += partial
    cnt_ref[...] += pcnt


def _readout_partials(f, gid3, W1, b1, W2, b2):
    blk = 512
    n = f.shape[0]
    return pl.pallas_call(
        _ro_body,
        grid=(n // blk,),
        in_specs=[
            pl.BlockSpec((blk, _H), lambda i: (i, 0)),
            pl.BlockSpec((1, 1, blk), lambda i: (i, 0, 0)),
            pl.BlockSpec((_H, _H), lambda i: (0, 0)),
            pl.BlockSpec((1, _H), lambda i: (0, 0)),
            pl.BlockSpec((_H, _H), lambda i: (0, 0)),
            pl.BlockSpec((1, _H), lambda i: (0, 0)),
        ],
        out_specs=[
            pl.BlockSpec((_G, _H), lambda i: (0, 0)),
            pl.BlockSpec((_G, _H), lambda i: (0, 0)),
        ],
        out_shape=[
            jax.ShapeDtypeStruct((_G, _H), jnp.float32),
            jax.ShapeDtypeStruct((_G, _H), jnp.float32),
        ],
    )(f, gid3, W1, b1, W2, b2)


def _fin_body(gs_ref, cnt_ref, wa_ref, wb_ref, b1_ref, w2_ref, b2_ref, o_ref):
    s = gs_ref[...]
    m = s / jnp.maximum(cnt_ref[...], 1.0)
    h = jnp.maximum(
        jnp.dot(s, wa_ref[...], preferred_element_type=jnp.float32)
        + jnp.dot(m, wb_ref[...], preferred_element_type=jnp.float32)
        + b1_ref[...], 0.0)
    o_ref[...] = (jnp.dot(h, w2_ref[...], preferred_element_type=jnp.float32)
                  + b2_ref[...])


def _final(gs, cnt, roA, roB, b1, W2, b2):
    return pl.pallas_call(
        _fin_body,
        in_specs=[
            pl.BlockSpec((_G, _H), lambda: (0, 0)),
            pl.BlockSpec((_G, _H), lambda: (0, 0)),
            pl.BlockSpec((_H, _H), lambda: (0, 0)),
            pl.BlockSpec((_H, _H), lambda: (0, 0)),
            pl.BlockSpec((1, _H), lambda: (0, 0)),
            pl.BlockSpec((_H, _T), lambda: (0, 0)),
            pl.BlockSpec((1, _T), lambda: (0, 0)),
        ],
        out_specs=pl.BlockSpec((_G, _T), lambda: (0, 0)),
        out_shape=jax.ShapeDtypeStruct((_G, _T), jnp.float32),
    )(gs, cnt, roA, roB, b1, W2, b2)


def _mesh():
    return plsc.VectorSubcoreMesh(core_axis_name="c", subcore_axis_name="s")


def _sc_gather(P, Q, src3, dst3):
    nwin = src3.shape[1]
    perw = nwin * _WIN
    ne = _NW * perw

    @functools.partial(
        pl.kernel,
        out_type=[
            jax.ShapeDtypeStruct((ne, _H), jnp.float32),
            jax.ShapeDtypeStruct((ne, _H), jnp.float32),
        ],
        mesh=_mesh(),
        scratch_types=(
            [pltpu.VMEM((nwin, _WIN), jnp.int32)] * 2
            + [pltpu.VMEM((_WIN, _H), jnp.float32)] * 4
            + [pltpu.SemaphoreType.DMA] * 4),
    )
    def k(p_hbm, q_hbm, s_hbm, d_hbm, op_hbm, oq_hbm,
          idxs, idxd, bufp0, bufq0, bufp1, bufq1, gsem0, gsem1, osem0, osem1):
        wid = lax.axis_index("s") * 2 + lax.axis_index("c")
        pltpu.sync_copy(s_hbm.at[wid], idxs)
        pltpu.sync_copy(d_hbm.at[wid], idxd)
        base = wid * perw
        slots = ((bufp0, bufq0, gsem0, osem0), (bufp1, bufq1, gsem1, osem1))

        def fire_gather(w, bp, bq, gs):
            pltpu.async_copy(p_hbm.at[idxs.at[w]], bp, gs)
            pltpu.async_copy(q_hbm.at[idxd.at[w]], bq, gs)

        def wait_sem(sem, ref):
            pltpu.make_async_copy(p_hbm.at[pl.ds(0, _WIN)], ref, sem).wait()

        fire_gather(0, bufp0, bufq0, gsem0)
        fire_gather(1, bufp1, bufq1, gsem1)

        @pl.loop(0, nwin, step=2)
        def _(j):
            for b in range(2):
                bp, bq, gs, os = slots[b]
                w = j + b
                wait_sem(gs, bp)
                wait_sem(gs, bq)
                off = base + w * _WIN
                pltpu.async_copy(bp, op_hbm.at[pl.ds(off, _WIN)], os)
                pltpu.async_copy(bq, oq_hbm.at[pl.ds(off, _WIN)], os)
            for b in range(2):
                bp, bq, gs, os = slots[b]
                wait_sem(os, bp)
                wait_sem(os, bq)

                @pl.when(j + 2 + b < nwin)
                def _():
                    fire_gather(j + 2 + b, bp, bq, gs)

    return k(P, Q, src3, dst3)


def _sc_scatter(M, dstS3, init):
    nwin = dstS3.shape[1]
    perw = nwin * _WIN

    @functools.partial(
        pl.kernel,
        out_type=jax.ShapeDtypeStruct((2, _ACC, _H), jnp.float32),
        mesh=_mesh(),
        scratch_types=(
            [pltpu.VMEM((nwin, _WIN), jnp.int32)]
            + [pltpu.VMEM((_WIN, _H), jnp.float32)] * 2
            + [pltpu.VMEM_SHARED((_ACC, _H), jnp.float32)]
            + [pltpu.SemaphoreType.DMA] * 2),
    )
    def k(m_hbm, d_hbm, i_hbm, out_hbm, idxd, mbuf0, mbuf1, acc, rs0, rs1):
        c = lax.axis_index("c")
        s = lax.axis_index("s")
        wid = s * 2 + c
        pltpu.sync_copy(i_hbm.at[c, pl.ds(s * _ZR, _ZR)],
                        acc.at[pl.ds(s * _ZR, _ZR)])
        pltpu.sync_copy(d_hbm.at[wid], idxd)
        plsc.subcore_barrier()
        base = wid * perw
        slots = ((mbuf0, rs0), (mbuf1, rs1))

        def fire_read(w, mb, rs):
            pltpu.async_copy(m_hbm.at[pl.ds(base + w * _WIN, _WIN)], mb, rs)

        fire_read(0, mbuf0, rs0)
        fire_read(1, mbuf1, rs1)

        @pl.loop(0, nwin, step=2)
        def _(j):
            for b in range(2):
                mb, rs = slots[b]
                w = j + b
                pltpu.make_async_copy(
                    m_hbm.at[pl.ds(0, _WIN)], mb, rs).wait()
                pltpu.sync_copy(mb, acc.at[idxd.at[w]], add=True)

                @pl.when(w + 2 < nwin)
                def _():
                    fire_read(w + 2, mb, rs)

        plsc.subcore_barrier()
        pltpu.sync_copy(acc.at[pl.ds(s * _ZR, _ZR)],
                        out_hbm.at[c, pl.ds(s * _ZR, _ZR)])

    return k(M, dstS3, init)


def kernel(x, edge_index, edge_w, graph_ids, W_in, b_in, msgW1, msgb1,
           msgW2, msgb2, updW1, updb1, updW2, updb2, outW1, outb1,
           outW2, outb2, roW1, rob1, roW2, rob2):
    src = edge_index[0].astype(jnp.int32)
    dst = edge_index[1].astype(jnp.int32)

    xp = jnp.pad(x, ((0, _NPAD - _N), (0, 0)))
    gid3 = jnp.pad(graph_ids.astype(jnp.int32), (0, _NPAD - _N),
                   constant_values=_G).reshape(_NPAD // 512, 1, 512)
    nch = 2
    csz = _EPAD // nch
    cwin = _NWIN // nch
    srcp = jnp.pad(src, (0, _EPAD - _E))
    dstp = jnp.pad(dst, (0, _EPAD - _E))
    dstSp = jnp.pad(dst, (0, _EPAD - _E), constant_values=_NPAD)
    ewp = jnp.pad(edge_w, ((0, _EPAD - _E), (0, 0)))
    src3 = [srcp[k * csz:(k + 1) * csz].reshape(_NW, cwin, _WIN)
            for k in range(nch)]
    dst3 = [dstp[k * csz:(k + 1) * csz].reshape(_NW, cwin, _WIN)
            for k in range(nch)]
    dstS3 = [dstSp[k * csz:(k + 1) * csz].reshape(_NW, cwin, _WIN)
             for k in range(nch)]
    ewc = [ewp[k * csz:(k + 1) * csz] for k in range(nch)]
    zinit = jnp.zeros((2, _ACC, _H), jnp.float32)

    f = _node_in(xp, W_in, b_in.reshape(1, _H))
    Wa = [msgW1[l, :_H] for l in range(_L)]
    Wb = [msgW1[l, _H:2 * _H] for l in range(_L)]
    bm = [msgb1[l].reshape(1, _H) for l in range(_L)]
    P, Q = _pq(f, Wa[0], Wb[0], bm[0])
    for l in range(_L):
        wc = msgW1[l, 2 * _H].reshape(1, _H)
        b2 = msgb2[l].reshape(1, _H)
        Pg0, Qg0 = _sc_gather(P, Q, src3[0], dst3[0])
        M0 = _edge(Pg0, Qg0, ewc[0], wc, msgW2[l], b2)
        Pg1, Qg1 = _sc_gather(P, Q, src3[1], dst3[1])
        partsA = _sc_scatter(M0, dstS3[0], zinit)
        M1 = _edge(Pg1, Qg1, ewc[1], wc, msgW2[l], b2)
        partsB = _sc_scatter(M1, dstS3[1], zinit)
        ln = min(l + 1, _L - 1)
        f, P, Q = _upd(f, partsA, partsB, updW1[l],
                       updb1[l].reshape(1, _H), updW2[l],
                       updb2[l].reshape(1, _H), Wa[ln], Wb[ln], bm[ln])

    gs, cnt = _readout_partials(f, gid3, outW1, outb1.reshape(1, _H),
                                outW2, outb2.reshape(1, _H))
    return _final(gs, cnt, roW1[:_H], roW1[_H:], rob1.reshape(1, _H),
                  roW2, rob2.reshape(1, _T))


# speedup vs baseline: 1.0900x; 1.0900x over previous
import functools

import jax
import jax.numpy as jnp
from jax import lax
from jax.experimental import pallas as pl
from jax.experimental.pallas import tpu as pltpu
from jax.experimental.pallas import tpu_sc as plsc

_N = 10000
_E = 320000
_H = 128
_G = 64
_T = 32
_L = 4

_NPAD = 10240
_NW = 32
_WIN = 128
_NWIN = 80
_PERW = _NWIN * _WIN
_EPAD = _NW * _PERW
_ACC = 10368
_ZR = _ACC // 16


def _silu(v):
    return v * jax.nn.sigmoid(v)


def _in_body(x_ref, w_ref, b_ref, o_ref):
    o_ref[...] = _silu(
        jnp.dot(x_ref[...], w_ref[...], preferred_element_type=jnp.float32)
        + b_ref[...])


def _node_in(x, W, b):
    blk = 512
    n = x.shape[0]
    return pl.pallas_call(
        _in_body,
        grid=(n // blk,),
        in_specs=[
            pl.BlockSpec((blk, _H), lambda i: (i, 0)),
            pl.BlockSpec((_H, _H), lambda i: (0, 0)),
            pl.BlockSpec((1, _H), lambda i: (0, 0)),
        ],
        out_specs=pl.BlockSpec((blk, _H), lambda i: (i, 0)),
        out_shape=jax.ShapeDtypeStruct((n, _H), jnp.float32),
    )(x, W, b)


def _pq_body(f_ref, wa_ref, wb_ref, b_ref, p_ref, q_ref):
    fb = f_ref[...]
    p_ref[...] = jnp.dot(fb, wa_ref[...], preferred_element_type=jnp.float32)
    q_ref[...] = (jnp.dot(fb, wb_ref[...], preferred_element_type=jnp.float32)
                  + b_ref[...])


def _pq(f, Wa, Wb, b1):
    blk = 512
    n = f.shape[0]
    return pl.pallas_call(
        _pq_body,
        grid=(n // blk,),
        in_specs=[
            pl.BlockSpec((blk, _H), lambda i: (i, 0)),
            pl.BlockSpec((_H, _H), lambda i: (0, 0)),
            pl.BlockSpec((_H, _H), lambda i: (0, 0)),
            pl.BlockSpec((1, _H), lambda i: (0, 0)),
        ],
        out_specs=[
            pl.BlockSpec((blk, _H), lambda i: (i, 0)),
            pl.BlockSpec((blk, _H), lambda i: (i, 0)),
        ],
        out_shape=[
            jax.ShapeDtypeStruct((n, _H), jnp.float32),
            jax.ShapeDtypeStruct((n, _H), jnp.float32),
        ],
    )(f, Wa, Wb, b1)


def _edge_body(pg_ref, qg_ref, ew_ref, wc_ref, w2_ref, b2_ref, o_ref):
    sq = ew_ref[...] ** 2
    m1 = _silu(pg_ref[...] + qg_ref[...] + sq * wc_ref[...])
    m2 = _silu(
        jnp.dot(m1, w2_ref[...], preferred_element_type=jnp.float32)
        + b2_ref[...])
    o_ref[...] = _silu(m2)


def _edge(Pg, Qg, ew, wc, W2, b2):
    blk = 512
    ne = Pg.shape[0]
    return pl.pallas_call(
        _edge_body,
        grid=(ne // blk,),
        in_specs=[
            pl.BlockSpec((blk, _H), lambda i: (i, 0)),
            pl.BlockSpec((blk, _H), lambda i: (i, 0)),
            pl.BlockSpec((blk, 1), lambda i: (i, 0)),
            pl.BlockSpec((1, _H), lambda i: (0, 0)),
            pl.BlockSpec((_H, _H), lambda i: (0, 0)),
            pl.BlockSpec((1, _H), lambda i: (0, 0)),
        ],
        out_specs=pl.BlockSpec((blk, _H), lambda i: (i, 0)),
        out_shape=jax.ShapeDtypeStruct((ne, _H), jnp.float32),
    )(Pg, Qg, ew, wc, W2, b2)


def _upd_body(f_ref, a0_ref, a1_ref, w1_ref, b1_ref,
              w2_ref, b2_ref, wa_ref, wb_ref, bm_ref,
              o_ref, p_ref, q_ref):
    fb = f_ref[...]
    u = a0_ref[0] + a1_ref[0] + fb
    h = _silu(
        jnp.dot(u, w1_ref[...], preferred_element_type=jnp.float32)
        + b1_ref[...])
    fn = fb + (
        jnp.dot(h, w2_ref[...], preferred_element_type=jnp.float32)
        + b2_ref[...])
    o_ref[...] = fn
    p_ref[...] = jnp.dot(fn, wa_ref[...], preferred_element_type=jnp.float32)
    q_ref[...] = (jnp.dot(fn, wb_ref[...], preferred_element_type=jnp.float32)
                  + bm_ref[...])


def _upd(f, parts, W1, b1, W2, b2, Wa, Wb, bm):
    blk = 512
    n = f.shape[0]
    return pl.pallas_call(
        _upd_body,
        grid=(n // blk,),
        in_specs=[
            pl.BlockSpec((blk, _H), lambda i: (i, 0)),
            pl.BlockSpec((1, blk, _H), lambda i: (0, i, 0)),
            pl.BlockSpec((1, blk, _H), lambda i: (1, i, 0)),
            pl.BlockSpec((_H, _H), lambda i: (0, 0)),
            pl.BlockSpec((1, _H), lambda i: (0, 0)),
            pl.BlockSpec((_H, _H), lambda i: (0, 0)),
            pl.BlockSpec((1, _H), lambda i: (0, 0)),
            pl.BlockSpec((_H, _H), lambda i: (0, 0)),
            pl.BlockSpec((_H, _H), lambda i: (0, 0)),
            pl.BlockSpec((1, _H), lambda i: (0, 0)),
        ],
        out_specs=[
            pl.BlockSpec((blk, _H), lambda i: (i, 0)),
            pl.BlockSpec((blk, _H), lambda i: (i, 0)),
            pl.BlockSpec((blk, _H), lambda i: (i, 0)),
        ],
        out_shape=[
            jax.ShapeDtypeStruct((n, _H), jnp.float32),
            jax.ShapeDtypeStruct((n, _H), jnp.float32),
            jax.ShapeDtypeStruct((n, _H), jnp.float32),
        ],
    )(f, parts, parts, W1, b1, W2, b2, Wa, Wb, bm)


def _ro_body(f_ref, gid_ref, w1_ref, b1_ref, w2_ref, b2_ref, gs_ref, cnt_ref):
    i = pl.program_id(0)
    f2 = _silu(
        jnp.dot(f_ref[...], w1_ref[...], preferred_element_type=jnp.float32)
        + b1_ref[...])
    f2 = (jnp.dot(f2, w2_ref[...], preferred_element_type=jnp.float32)
          + b2_ref[...])
    ids = gid_ref[0, 0, :]
    blk = ids.shape[0]
    onehot = (ids[:, None]
              == lax.broadcasted_iota(jnp.int32, (blk, _G), 1)
              ).astype(jnp.float32)
    partial = lax.dot_general(onehot, f2, (((0,), (0,)), ((), ())),
                              preferred_element_type=jnp.float32)
    pcnt = lax.dot_general(onehot, jnp.ones((blk, _H), jnp.float32),
                           (((0,), (0,)), ((), ())),
                           preferred_element_type=jnp.float32)

    @pl.when(i == 0)
    def _():
        gs_ref[...] = jnp.zeros_like(gs_ref)
        cnt_ref[...] = jnp.zeros_like(cnt_ref)

    gs_ref[...] += partial
    cnt_ref[...] += pcnt


def _readout_partials(f, gid3, W1, b1, W2, b2):
    blk = 512
    n = f.shape[0]
    return pl.pallas_call(
        _ro_body,
        grid=(n // blk,),
        in_specs=[
            pl.BlockSpec((blk, _H), lambda i: (i, 0)),
            pl.BlockSpec((1, 1, blk), lambda i: (i, 0, 0)),
            pl.BlockSpec((_H, _H), lambda i: (0, 0)),
            pl.BlockSpec((1, _H), lambda i: (0, 0)),
            pl.BlockSpec((_H, _H), lambda i: (0, 0)),
            pl.BlockSpec((1, _H), lambda i: (0, 0)),
        ],
        out_specs=[
            pl.BlockSpec((_G, _H), lambda i: (0, 0)),
            pl.BlockSpec((_G, _H), lambda i: (0, 0)),
        ],
        out_shape=[
            jax.ShapeDtypeStruct((_G, _H), jnp.float32),
            jax.ShapeDtypeStruct((_G, _H), jnp.float32),
        ],
    )(f, gid3, W1, b1, W2, b2)


def _fin_body(gs_ref, cnt_ref, wa_ref, wb_ref, b1_ref, w2_ref, b2_ref, o_ref):
    s = gs_ref[...]
    m = s / jnp.maximum(cnt_ref[...], 1.0)
    h = jnp.maximum(
        jnp.dot(s, wa_ref[...], preferred_element_type=jnp.float32)
        + jnp.dot(m, wb_ref[...], preferred_element_type=jnp.float32)
        + b1_ref[...], 0.0)
    o_ref[...] = (jnp.dot(h, w2_ref[...], preferred_element_type=jnp.float32)
                  + b2_ref[...])


def _final(gs, cnt, roA, roB, b1, W2, b2):
    return pl.pallas_call(
        _fin_body,
        in_specs=[
            pl.BlockSpec((_G, _H), lambda: (0, 0)),
            pl.BlockSpec((_G, _H), lambda: (0, 0)),
            pl.BlockSpec((_H, _H), lambda: (0, 0)),
            pl.BlockSpec((_H, _H), lambda: (0, 0)),
            pl.BlockSpec((1, _H), lambda: (0, 0)),
            pl.BlockSpec((_H, _T), lambda: (0, 0)),
            pl.BlockSpec((1, _T), lambda: (0, 0)),
        ],
        out_specs=pl.BlockSpec((_G, _T), lambda: (0, 0)),
        out_shape=jax.ShapeDtypeStruct((_G, _T), jnp.float32),
    )(gs, cnt, roA, roB, b1, W2, b2)


def _mesh():
    return plsc.VectorSubcoreMesh(core_axis_name="c", subcore_axis_name="s")


def _sc_gather(P, Q, src3, dst3):
    nwin = src3.shape[1]
    perw = nwin * _WIN
    ne = _NW * perw

    @functools.partial(
        pl.kernel,
        out_type=[
            jax.ShapeDtypeStruct((ne, _H), jnp.float32),
            jax.ShapeDtypeStruct((ne, _H), jnp.float32),
        ],
        mesh=_mesh(),
        scratch_types=(
            [pltpu.VMEM((nwin, _WIN), jnp.int32)] * 2
            + [pltpu.VMEM((_WIN, _H), jnp.float32)] * 4
            + [pltpu.SemaphoreType.DMA] * 4),
    )
    def k(p_hbm, q_hbm, s_hbm, d_hbm, op_hbm, oq_hbm,
          idxs, idxd, bufp0, bufq0, bufp1, bufq1, gsem0, gsem1, osem0, osem1):
        wid = lax.axis_index("s") * 2 + lax.axis_index("c")
        pltpu.sync_copy(s_hbm.at[wid], idxs)
        pltpu.sync_copy(d_hbm.at[wid], idxd)
        base = wid * perw
        slots = ((bufp0, bufq0, gsem0, osem0), (bufp1, bufq1, gsem1, osem1))

        def fire_gather(w, bp, bq, gs):
            pltpu.async_copy(p_hbm.at[idxs.at[w]], bp, gs)
            pltpu.async_copy(q_hbm.at[idxd.at[w]], bq, gs)

        def wait_sem(sem, ref):
            pltpu.make_async_copy(p_hbm.at[pl.ds(0, _WIN)], ref, sem).wait()

        fire_gather(0, bufp0, bufq0, gsem0)
        fire_gather(1, bufp1, bufq1, gsem1)

        @pl.loop(0, nwin, step=2)
        def _(j):
            for b in range(2):
                bp, bq, gs, os = slots[b]
                w = j + b
                wait_sem(gs, bp)
                wait_sem(gs, bq)
                off = base + w * _WIN
                pltpu.async_copy(bp, op_hbm.at[pl.ds(off, _WIN)], os)
                pltpu.async_copy(bq, oq_hbm.at[pl.ds(off, _WIN)], os)
            for b in range(2):
                bp, bq, gs, os = slots[b]
                wait_sem(os, bp)
                wait_sem(os, bq)

                @pl.when(j + 2 + b < nwin)
                def _():
                    fire_gather(j + 2 + b, bp, bq, gs)

    return k(P, Q, src3, dst3)


def _sc_scatter(M, dstS3, init):
    nwin = dstS3.shape[1]
    perw = nwin * _WIN

    @functools.partial(
        pl.kernel,
        out_type=jax.ShapeDtypeStruct((2, _ACC, _H), jnp.float32),
        mesh=_mesh(),
        scratch_types=(
            [pltpu.VMEM((nwin, _WIN), jnp.int32)]
            + [pltpu.VMEM((_WIN, _H), jnp.float32)] * 2
            + [pltpu.VMEM_SHARED((_ACC, _H), jnp.float32)]
            + [pltpu.SemaphoreType.DMA] * 2),
    )
    def k(m_hbm, d_hbm, i_hbm, out_hbm, idxd, mbuf0, mbuf1, acc, rs0, rs1):
        c = lax.axis_index("c")
        s = lax.axis_index("s")
        wid = s * 2 + c
        pltpu.sync_copy(i_hbm.at[c, pl.ds(s * _ZR, _ZR)],
                        acc.at[pl.ds(s * _ZR, _ZR)])
        pltpu.sync_copy(d_hbm.at[wid], idxd)
        plsc.subcore_barrier()
        base = wid * perw
        slots = ((mbuf0, rs0), (mbuf1, rs1))

        def fire_read(w, mb, rs):
            pltpu.async_copy(m_hbm.at[pl.ds(base + w * _WIN, _WIN)], mb, rs)

        fire_read(0, mbuf0, rs0)
        fire_read(1, mbuf1, rs1)

        @pl.loop(0, nwin, step=2)
        def _(j):
            for b in range(2):
                mb, rs = slots[b]
                w = j + b
                pltpu.make_async_copy(
                    m_hbm.at[pl.ds(0, _WIN)], mb, rs).wait()
                pltpu.sync_copy(mb, acc.at[idxd.at[w]], add=True)

                @pl.when(w + 2 < nwin)
                def _():
                    fire_read(w + 2, mb, rs)

        plsc.subcore_barrier()
        pltpu.sync_copy(acc.at[pl.ds(s * _ZR, _ZR)],
                        out_hbm.at[c, pl.ds(s * _ZR, _ZR)])

    return k(M, dstS3, init)


def kernel(x, edge_index, edge_w, graph_ids, W_in, b_in, msgW1, msgb1,
           msgW2, msgb2, updW1, updb1, updW2, updb2, outW1, outb1,
           outW2, outb2, roW1, rob1, roW2, rob2):
    src = edge_index[0].astype(jnp.int32)
    dst = edge_index[1].astype(jnp.int32)

    xp = jnp.pad(x, ((0, _NPAD - _N), (0, 0)))
    gid3 = jnp.pad(graph_ids.astype(jnp.int32), (0, _NPAD - _N),
                   constant_values=_G).reshape(_NPAD // 512, 1, 512)
    nch = 2
    csz = _EPAD // nch
    cwin = _NWIN // nch
    srcp = jnp.pad(src, (0, _EPAD - _E))
    dstp = jnp.pad(dst, (0, _EPAD - _E))
    dstSp = jnp.pad(dst, (0, _EPAD - _E), constant_values=_NPAD)
    ewp = jnp.pad(edge_w, ((0, _EPAD - _E), (0, 0)))
    src3 = [srcp[k * csz:(k + 1) * csz].reshape(_NW, cwin, _WIN)
            for k in range(nch)]
    dst3 = [dstp[k * csz:(k + 1) * csz].reshape(_NW, cwin, _WIN)
            for k in range(nch)]
    dstS3 = [dstSp[k * csz:(k + 1) * csz].reshape(_NW, cwin, _WIN)
             for k in range(nch)]
    ewc = [ewp[k * csz:(k + 1) * csz] for k in range(nch)]
    zinit = jnp.zeros((2, _ACC, _H), jnp.float32)

    f = _node_in(xp, W_in, b_in.reshape(1, _H))
    Wa = [msgW1[l, :_H] for l in range(_L)]
    Wb = [msgW1[l, _H:2 * _H] for l in range(_L)]
    bm = [msgb1[l].reshape(1, _H) for l in range(_L)]
    P, Q = _pq(f, Wa[0], Wb[0], bm[0])
    for l in range(_L):
        wc = msgW1[l, 2 * _H].reshape(1, _H)
        b2 = msgb2[l].reshape(1, _H)
        Pg0, Qg0 = _sc_gather(P, Q, src3[0], dst3[0])
        M0 = _edge(Pg0, Qg0, ewc[0], wc, msgW2[l], b2)
        Pg1, Qg1 = _sc_gather(P, Q, src3[1], dst3[1])
        partsA = _sc_scatter(M0, dstS3[0], zinit)
        M1 = _edge(Pg1, Qg1, ewc[1], wc, msgW2[l], b2)
        partsB = _sc_scatter(M1, dstS3[1], partsA)
        ln = min(l + 1, _L - 1)
        f, P, Q = _upd(f, partsB, updW1[l],
                       updb1[l].reshape(1, _H), updW2[l],
                       updb2[l].reshape(1, _H), Wa[ln], Wb[ln], bm[ln])

    gs, cnt = _readout_partials(f, gid3, outW1, outb1.reshape(1, _H),
                                outW2, outb2.reshape(1, _H))
    return _final(gs, cnt, roW1[:_H], roW1[_H:], rob1.reshape(1, _H),
                  roW2, rob2.reshape(1, _T))
